# baseline (device time: 37140 ns/iter reference)
import jax
import jax.numpy as jnp
from jax import lax
from jax.experimental import pallas as pl
from jax.experimental.pallas import tpu as pltpu

N_DEV = 8
CW_HOPS = 4
CCW_HOPS = 3


def kernel(x, router_W, route_idx, expert_W, shared_W):
    n_tok, d_model = x.shape
    e_per, _, d_ff = expert_W.shape
    n_exp = N_DEV * e_per

    def body(x_ref, rw_ref, idx_ref, ew_ref, sw_ref, out_ref,
             ewb_ref, xg_ref, cw_ref, ccw_ref,
             cw_send, cw_recv, ccw_send, ccw_recv):
        my = lax.axis_index("i")
        left = lax.rem(my + N_DEV - 1, N_DEV)
        right = lax.rem(my + 1, N_DEV)

        barrier_sem = pltpu.get_barrier_semaphore()
        for nbr in (left, right):
            pl.semaphore_signal(
                barrier_sem, inc=1,
                device_id=(nbr,), device_id_type=pl.DeviceIdType.MESH,
            )
        pl.semaphore_wait(barrier_sem, 2)

        xv = x_ref[...]
        ewb_ref[...] = ew_ref[...].astype(jnp.bfloat16)
        scores = jnp.dot(xv, rw_ref[...], preferred_element_type=jnp.float32)
        s_max = jnp.max(scores, axis=-1, keepdims=True)
        ex = jnp.exp(scores - s_max)
        probs = ex / jnp.sum(ex, axis=-1, keepdims=True)
        route = idx_ref[...]
        eids = lax.broadcasted_iota(jnp.int32, (n_tok, n_exp), 1)
        gates = jnp.where(route == eids, probs, 0.0)
        xg_ref[...] = (
            xv[:, None, :] * gates[:, :, None]
        ).astype(jnp.bfloat16).reshape(n_tok, n_exp * d_model)

        def make_chain(comm_ref, send_sems, recv_sems, hops, target):
            return [
                [
                    pltpu.make_async_remote_copy(
                        src_ref=(ewb_ref.at[k] if h == 0
                                 else comm_ref.at[h, k]),
                        dst_ref=comm_ref.at[h + 1, k],
                        send_sem=send_sems.at[h, k],
                        recv_sem=recv_sems.at[h + 1, k],
                        device_id=(target,),
                        device_id_type=pl.DeviceIdType.MESH,
                    )
                    for k in range(e_per)
                ]
                for h in range(hops)
            ]

        cw_fwd = make_chain(cw_ref, cw_send, cw_recv, CW_HOPS, right)
        ccw_fwd = make_chain(ccw_ref, ccw_send, ccw_recv, CCW_HOPS, left)

        for k in range(e_per):
            cw_fwd[0][k].start()
            ccw_fwd[0][k].start()

        def compute_block(block, src_dev):
            xg = xg_ref[:, pl.ds(src_dev * e_per * d_model,
                                 e_per * d_model)]
            out_ref[...] += jnp.dot(
                xg, block[...].reshape(e_per * d_model, d_ff),
                preferred_element_type=jnp.float32,
            )

        out_ref[...] = jnp.dot(
            xv.astype(jnp.bfloat16), sw_ref[...].astype(jnp.bfloat16),
            preferred_element_type=jnp.float32,
        )
        compute_block(ewb_ref, my)

        def recv_and_forward(chain, r, hops):
            for k in range(e_per):
                chain[r - 1][k].wait_recv()
                if r < hops:
                    chain[r][k].start()

        for r in range(1, CW_HOPS + 1):
            recv_and_forward(cw_fwd, r, CW_HOPS)
            if r <= CCW_HOPS:
                recv_and_forward(ccw_fwd, r, CCW_HOPS)
            compute_block(cw_ref.at[r], lax.rem(my - r + N_DEV, N_DEV))
            if r <= CCW_HOPS:
                compute_block(ccw_ref.at[r], lax.rem(my + r, N_DEV))

        for chain in (cw_fwd, ccw_fwd):
            for hop in chain:
                for d in hop:
                    d.wait_send()

    return pl.pallas_call(
        body,
        out_shape=jax.ShapeDtypeStruct((n_tok, d_ff), jnp.float32),
        in_specs=[pl.BlockSpec(memory_space=pltpu.VMEM)] * 5,
        out_specs=pl.BlockSpec(memory_space=pltpu.VMEM),
        scratch_shapes=[
            pltpu.VMEM((e_per, d_model, d_ff), jnp.bfloat16),
            pltpu.VMEM((n_tok, N_DEV * e_per * d_model), jnp.bfloat16),
            pltpu.VMEM((CW_HOPS + 1, e_per, d_model, d_ff), jnp.bfloat16),
            pltpu.VMEM((CCW_HOPS + 1, e_per, d_model, d_ff), jnp.bfloat16),
            pltpu.SemaphoreType.DMA((CW_HOPS, e_per)),
            pltpu.SemaphoreType.DMA((CW_HOPS + 1, e_per)),
            pltpu.SemaphoreType.DMA((CCW_HOPS, e_per)),
            pltpu.SemaphoreType.DMA((CCW_HOPS + 1, e_per)),
        ],
        compiler_params=pltpu.CompilerParams(collective_id=0),
    )(x, router_W, route_idx, expert_W, shared_W)


# device time: 34826 ns/iter; 1.0664x vs baseline; 1.0664x over previous
import jax
import jax.numpy as jnp
from jax import lax
from jax.experimental import pallas as pl
from jax.experimental.pallas import tpu as pltpu

N_DEV = 8
CW_HOPS = 4
CCW_HOPS = 3


def kernel(x, router_W, route_idx, expert_W, shared_W):
    n_tok, d_model = x.shape
    e_per, _, d_ff = expert_W.shape
    n_exp = N_DEV * e_per

    def body(x_ref, rw_ref, idx_ref, ew_ref, sw_ref, out_ref,
             ewb_ref, xg_ref, cw_ref, ccw_ref,
             cw_send, cw_recv, ccw_send, ccw_recv):
        my = lax.axis_index("i")
        left = lax.rem(my + N_DEV - 1, N_DEV)
        right = lax.rem(my + 1, N_DEV)

        barrier_sem = pltpu.get_barrier_semaphore()
        for nbr in (left, right):
            pl.semaphore_signal(
                barrier_sem, inc=1,
                device_id=(nbr,), device_id_type=pl.DeviceIdType.MESH,
            )
        pl.semaphore_wait(barrier_sem, 2)

        xv = x_ref[...]
        ewb_ref[...] = ew_ref[...].astype(jnp.bfloat16)

        def make_chain(comm_ref, send_sems, recv_sems, hops, target):
            return [
                [
                    pltpu.make_async_remote_copy(
                        src_ref=(ewb_ref.at[k] if h == 0
                                 else comm_ref.at[h, k]),
                        dst_ref=comm_ref.at[h + 1, k],
                        send_sem=send_sems.at[h, k],
                        recv_sem=recv_sems.at[h + 1, k],
                        device_id=(target,),
                        device_id_type=pl.DeviceIdType.MESH,
                    )
                    for k in range(e_per)
                ]
                for h in range(hops)
            ]

        cw_fwd = make_chain(cw_ref, cw_send, cw_recv, CW_HOPS, right)
        ccw_fwd = make_chain(ccw_ref, ccw_send, ccw_recv, CCW_HOPS, left)

        for k in range(e_per):
            cw_fwd[0][k].start()
            ccw_fwd[0][k].start()

        scores = jnp.dot(xv, rw_ref[...], preferred_element_type=jnp.float32)
        s_max = jnp.max(scores, axis=-1, keepdims=True)
        ex = jnp.exp(scores - s_max)
        probs = ex / jnp.sum(ex, axis=-1, keepdims=True)
        route = idx_ref[...]
        eids = lax.broadcasted_iota(jnp.int32, (n_tok, n_exp), 1)
        gates = jnp.where(route == eids, probs, 0.0)
        xg_ref[...] = (
            xv[:, None, :] * gates[:, :, None]
        ).astype(jnp.bfloat16).reshape(n_tok, n_exp * d_model)

        def compute_block(block, src_dev):
            xg = xg_ref[:, pl.ds(src_dev * e_per * d_model,
                                 e_per * d_model)]
            out_ref[...] += jnp.dot(
                xg, block[...].reshape(e_per * d_model, d_ff),
                preferred_element_type=jnp.float32,
            )

        out_ref[...] = jnp.dot(
            xv.astype(jnp.bfloat16), sw_ref[...].astype(jnp.bfloat16),
            preferred_element_type=jnp.float32,
        )
        compute_block(ewb_ref, my)

        def recv_and_forward(chain, r, hops):
            for k in range(e_per):
                chain[r - 1][k].wait_recv()
                if r < hops:
                    chain[r][k].start()

        for r in range(1, CW_HOPS + 1):
            recv_and_forward(cw_fwd, r, CW_HOPS)
            if r <= CCW_HOPS:
                recv_and_forward(ccw_fwd, r, CCW_HOPS)
            compute_block(cw_ref.at[r], lax.rem(my - r + N_DEV, N_DEV))
            if r <= CCW_HOPS:
                compute_block(ccw_ref.at[r], lax.rem(my + r, N_DEV))

        for chain in (cw_fwd, ccw_fwd):
            for hop in chain:
                for d in hop:
                    d.wait_send()

    return pl.pallas_call(
        body,
        out_shape=jax.ShapeDtypeStruct((n_tok, d_ff), jnp.float32),
        in_specs=[pl.BlockSpec(memory_space=pltpu.VMEM)] * 5,
        out_specs=pl.BlockSpec(memory_space=pltpu.VMEM),
        scratch_shapes=[
            pltpu.VMEM((e_per, d_model, d_ff), jnp.bfloat16),
            pltpu.VMEM((n_tok, N_DEV * e_per * d_model), jnp.bfloat16),
            pltpu.VMEM((CW_HOPS + 1, e_per, d_model, d_ff), jnp.bfloat16),
            pltpu.VMEM((CCW_HOPS + 1, e_per, d_model, d_ff), jnp.bfloat16),
            pltpu.SemaphoreType.DMA((CW_HOPS, e_per)),
            pltpu.SemaphoreType.DMA((CW_HOPS + 1, e_per)),
            pltpu.SemaphoreType.DMA((CCW_HOPS, e_per)),
            pltpu.SemaphoreType.DMA((CCW_HOPS + 1, e_per)),
        ],
        compiler_params=pltpu.CompilerParams(collective_id=0),
    )(x, router_W, route_idx, expert_W, shared_W)


# device time: 33952 ns/iter; 1.0939x vs baseline; 1.0257x over previous
import jax
import jax.numpy as jnp
from jax import lax
from jax.experimental import pallas as pl
from jax.experimental.pallas import tpu as pltpu

N_DEV = 8
CW_HOPS = 3
CCW_HOPS = 3


def kernel(x, router_W, route_idx, expert_W, shared_W):
    n_tok, d_model = x.shape
    e_per, _, d_ff = expert_W.shape
    n_exp = N_DEV * e_per

    def body(x_ref, rw_ref, idx_ref, ew_ref, sw_ref, out_ref,
             ewb_ref, xg_ref, cw_ref, ccw_ref, z_ref,
             cw_send, cw_recv, ccw_send, ccw_recv, z_send, z_recv):
        my = lax.axis_index("i")
        left = lax.rem(my + N_DEV - 1, N_DEV)
        right = lax.rem(my + 1, N_DEV)
        zpart = lax.rem(my + N_DEV // 2, N_DEV)

        barrier_sem = pltpu.get_barrier_semaphore()
        for nbr in (left, right, zpart):
            pl.semaphore_signal(
                barrier_sem, inc=1,
                device_id=(nbr,), device_id_type=pl.DeviceIdType.MESH,
            )
        pl.semaphore_wait(barrier_sem, 3)

        xv = x_ref[...]
        ewb_ref[...] = ew_ref[...].astype(jnp.bfloat16)

        def make_chain(comm_ref, send_sems, recv_sems, hops, target):
            return [
                [
                    pltpu.make_async_remote_copy(
                        src_ref=(ewb_ref.at[k] if h == 0
                                 else comm_ref.at[h, k]),
                        dst_ref=comm_ref.at[h + 1, k],
                        send_sem=send_sems.at[h, k],
                        recv_sem=recv_sems.at[h + 1, k],
                        device_id=(target,),
                        device_id_type=pl.DeviceIdType.MESH,
                    )
                    for k in range(e_per)
                ]
                for h in range(hops)
            ]

        cw_fwd = make_chain(cw_ref, cw_send, cw_recv, CW_HOPS, right)
        ccw_fwd = make_chain(ccw_ref, ccw_send, ccw_recv, CCW_HOPS, left)
        z_xfer = pltpu.make_async_remote_copy(
            src_ref=ewb_ref,
            dst_ref=z_ref,
            send_sem=z_send,
            recv_sem=z_recv,
            device_id=(zpart,),
            device_id_type=pl.DeviceIdType.MESH,
        )

        z_xfer.start()
        for k in range(e_per):
            cw_fwd[0][k].start()
            ccw_fwd[0][k].start()

        scores = jnp.dot(xv, rw_ref[...], preferred_element_type=jnp.float32)
        s_max = jnp.max(scores, axis=-1, keepdims=True)
        ex = jnp.exp(scores - s_max)
        probs = ex / jnp.sum(ex, axis=-1, keepdims=True)
        route = idx_ref[...]
        eids = lax.broadcasted_iota(jnp.int32, (n_tok, n_exp), 1)
        gates = jnp.where(route == eids, probs, 0.0)
        xg_ref[...] = (
            xv[:, None, :] * gates[:, :, None]
        ).astype(jnp.bfloat16).reshape(n_tok, n_exp * d_model)

        def compute_block(block, src_dev):
            xg = xg_ref[:, pl.ds(src_dev * e_per * d_model,
                                 e_per * d_model)]
            out_ref[...] += jnp.dot(
                xg, block[...].reshape(e_per * d_model, d_ff),
                preferred_element_type=jnp.float32,
            )

        out_ref[...] = jnp.dot(
            xv.astype(jnp.bfloat16), sw_ref[...].astype(jnp.bfloat16),
            preferred_element_type=jnp.float32,
        )
        compute_block(ewb_ref, my)

        def recv_and_forward(chain, r, hops):
            for k in range(e_per):
                chain[r - 1][k].wait_recv()
                if r < hops:
                    chain[r][k].start()

        for r in range(1, CW_HOPS + 1):
            recv_and_forward(cw_fwd, r, CW_HOPS)
            if r <= CCW_HOPS:
                recv_and_forward(ccw_fwd, r, CCW_HOPS)
            compute_block(cw_ref.at[r], lax.rem(my - r + N_DEV, N_DEV))
            if r <= CCW_HOPS:
                compute_block(ccw_ref.at[r], lax.rem(my + r, N_DEV))

        z_xfer.wait_recv()
        compute_block(z_ref, zpart)

        z_xfer.wait_send()
        for chain in (cw_fwd, ccw_fwd):
            for hop in chain:
                for d in hop:
                    d.wait_send()

    return pl.pallas_call(
        body,
        out_shape=jax.ShapeDtypeStruct((n_tok, d_ff), jnp.float32),
        in_specs=[pl.BlockSpec(memory_space=pltpu.VMEM)] * 5,
        out_specs=pl.BlockSpec(memory_space=pltpu.VMEM),
        scratch_shapes=[
            pltpu.VMEM((e_per, d_model, d_ff), jnp.bfloat16),
            pltpu.VMEM((n_tok, N_DEV * e_per * d_model), jnp.bfloat16),
            pltpu.VMEM((CW_HOPS + 1, e_per, d_model, d_ff), jnp.bfloat16),
            pltpu.VMEM((CCW_HOPS + 1, e_per, d_model, d_ff), jnp.bfloat16),
            pltpu.VMEM((e_per, d_model, d_ff), jnp.bfloat16),
            pltpu.SemaphoreType.DMA((CW_HOPS, e_per)),
            pltpu.SemaphoreType.DMA((CW_HOPS + 1, e_per)),
            pltpu.SemaphoreType.DMA((CCW_HOPS, e_per)),
            pltpu.SemaphoreType.DMA((CCW_HOPS + 1, e_per)),
            pltpu.SemaphoreType.DMA,
            pltpu.SemaphoreType.DMA,
        ],
        compiler_params=pltpu.CompilerParams(collective_id=0),
    )(x, router_W, route_idx, expert_W, shared_W)
